# baseline (speedup 1.0000x reference)
import jax
import jax.numpy as jnp
from jax.experimental import pallas as pl
from jax.experimental.pallas import tpu as pltpu

_N, _C, _H, _W = 8, 32, 128, 128
_Q = 16
_CH = 8


def _minmax_body(x_ref, o_ref):
    x = x_ref[...]
    o_ref[0] = jnp.min(x)
    o_ref[1] = jnp.max(x)


def _shift_w(y, d, lane, mask):
    if d == 0:
        return y
    r = pltpu.roll(y, (-d) % _W, axis=2)
    if not mask:
        return r
    edge = _W - 1 if d == 1 else 0
    return jnp.where(lane == edge, 0, r)


def _shift_h(y, d, sub, mask):
    if d == 0:
        return y
    r = pltpu.roll(y, (-d) % _H, axis=1)
    if not mask:
        return r
    edge = _H - 1 if d == 1 else 0
    return jnp.where(sub == edge, 0, r)


def _band(get, c0, nrows, dtype):
    lo = c0 - 1
    hi = lo + nrows
    parts = []
    if lo < 0:
        parts.append(jnp.zeros((-lo, _H, _W), dtype))
        lo = 0
    body = get(lo, min(hi, _C))
    parts.append(body)
    if hi > _C:
        parts.append(jnp.zeros((hi - _C, _H, _W), dtype))
    return jnp.concatenate(parts, axis=0) if len(parts) > 1 else parts[0]


def _main_body(mm_ref, f_ref, tab_ref, x_ref, o_ref, idx_ref):
    xmin = mm_ref[0]
    xmax = mm_ref[1]
    q = jnp.float32(_Q)

    x_all = x_ref[0]
    t = (x_all - xmin) / xmax * q
    idx_ref[...] = jnp.clip(jnp.floor(t).astype(jnp.int32), 0, _Q - 1)

    nb = _CH + 2
    lane = jax.lax.broadcasted_iota(jnp.int32, (nb, _H, _W), 2)
    sub = jax.lax.broadcasted_iota(jnp.int32, (nb, _H, _W), 1)
    tab = jnp.broadcast_to(tab_ref[0][None, None, :], (_CH, _H, _W))

    for c0 in range(0, _C, _CH):
        xband = _band(lambda a, b: x_ref[0, a:b], c0, nb, jnp.float32)
        bband = _band(lambda a, b: idx_ref[a:b], c0, nb, jnp.int32)

        a8 = idx_ref[c0:c0 + _CH] * (_Q // 2)
        r = [jnp.take_along_axis(tab, a8 + j, axis=2) for j in range(8)]

        acc = jnp.zeros((_CH, _H, _W), jnp.float32)
        for dw in (-1, 0, 1):
            xw = _shift_w(xband, dw, lane, True)
            bw = _shift_w(bband, dw, lane, False)
            for dh in (-1, 0, 1):
                xwh = _shift_h(xw, dh, sub, True)
                bwh = _shift_h(bw, dh, sub, False)
                for dc in (-1, 0, 1):
                    xs = xwh[1 + dc:1 + dc + _CH]
                    bs = bwh[1 + dc:1 + dc + _CH]
                    m0 = (bs & 2) != 0
                    s0 = jnp.where(m0, r[1], r[0])
                    s1 = jnp.where(m0, r[3], r[2])
                    s2 = jnp.where(m0, r[5], r[4])
                    s3 = jnp.where(m0, r[7], r[6])
                    m1 = (bs & 4) != 0
                    t0 = jnp.where(m1, s1, s0)
                    t1 = jnp.where(m1, s3, s2)
                    m2 = (bs & 8) != 0
                    u = jnp.where(m2, t1, t0)
                    odd = (bs & 1) == 1
                    bits = jnp.where(odd, u & jnp.int32(-65536), u << 16)
                    val = pltpu.bitcast(bits, jnp.float32)
                    ft = f_ref[(dc + 1) * 9 + (dh + 1) * 3 + (dw + 1)]
                    acc = acc + (ft * xs) * val
        o_ref[0, c0:c0 + _CH] = acc


def _pack_co_table(co_matrix):
    cb = co_matrix.reshape(-1).astype(jnp.bfloat16)
    u16 = jax.lax.bitcast_convert_type(cb, jnp.uint16).astype(jnp.uint32)
    packed = u16[0::2] | (u16[1::2] << 16)
    return packed.astype(jnp.int32).reshape(1, 128)


def kernel(x, co_matrix, spatial_filter):
    xr = x.reshape(_N * _C * _H, _W)
    mm = pl.pallas_call(
        _minmax_body,
        out_shape=jax.ShapeDtypeStruct((2,), jnp.float32),
        in_specs=[pl.BlockSpec(memory_space=pltpu.VMEM)],
        out_specs=pl.BlockSpec(memory_space=pltpu.SMEM),
    )(xr)

    tab = _pack_co_table(co_matrix)
    f = spatial_filter.reshape(27)

    out = pl.pallas_call(
        _main_body,
        grid=(_N,),
        out_shape=jax.ShapeDtypeStruct((_N, _C, _H, _W), jnp.float32),
        in_specs=[
            pl.BlockSpec(memory_space=pltpu.SMEM),
            pl.BlockSpec(memory_space=pltpu.SMEM),
            pl.BlockSpec((1, 128), lambda n: (0, 0)),
            pl.BlockSpec((1, _C, _H, _W), lambda n: (n, 0, 0, 0)),
        ],
        out_specs=pl.BlockSpec((1, _C, _H, _W), lambda n: (n, 0, 0, 0)),
        scratch_shapes=[pltpu.VMEM((_C, _H, _W), jnp.int32)],
        compiler_params=pltpu.CompilerParams(
            dimension_semantics=("parallel",),
        ),
    )(mm, f, tab, x)
    return out


# speedup vs baseline: 1.0505x; 1.0505x over previous
import jax
import jax.numpy as jnp
from jax.experimental import pallas as pl
from jax.experimental.pallas import tpu as pltpu

_N, _C, _H, _W = 8, 32, 128, 128
_Q = 16


def _minmax_body(x_ref, o_ref):
    x = x_ref[...]
    o_ref[0] = jnp.min(x)
    o_ref[1] = jnp.max(x)


def _shift_w(y, d, lane, mask):
    if d == 0:
        return y
    r = pltpu.roll(y, (-d) % _W, axis=2)
    if not mask:
        return r
    edge = _W - 1 if d == 1 else 0
    return jnp.where(lane == edge, 0, r)


def _shift_h(y, d, sub, mask):
    if d == 0:
        return y
    r = pltpu.roll(y, (-d) % _H, axis=1)
    if not mask:
        return r
    edge = _H - 1 if d == 1 else 0
    return jnp.where(sub == edge, 0, r)


def _shift_c(y, d, mask):
    if d == 0:
        return y
    if not mask:
        return pltpu.roll(y, (-d) % _C, axis=0)
    z = jnp.zeros((1, _H, _W), y.dtype)
    if d == 1:
        return jnp.concatenate([y[1:], z], axis=0)
    return jnp.concatenate([z, y[:-1]], axis=0)


def _main_body(mm_ref, f_ref, tab_ref, x_ref, o_ref):
    x = x_ref[0]
    xmin = mm_ref[0]
    xmax = mm_ref[1]
    q = jnp.float32(_Q)
    t = (x - xmin) / xmax * q
    idx = jnp.clip(jnp.floor(t).astype(jnp.int32), 0, _Q - 1)
    a8 = idx * (_Q // 2)

    lane = jax.lax.broadcasted_iota(jnp.int32, (_C, _H, _W), 2)
    sub = jax.lax.broadcasted_iota(jnp.int32, (_C, _H, _W), 1)

    tab = jnp.broadcast_to(tab_ref[0][None, None, :], (_C, _H, _W))

    acc = jnp.zeros((_C, _H, _W), jnp.float32)
    for dw in (-1, 0, 1):
        xw = _shift_w(x, dw, lane, True)
        bw = _shift_w(idx, dw, lane, True)
        for dh in (-1, 0, 1):
            xwh = _shift_h(xw, dh, sub, True)
            bwh = _shift_h(bw, dh, sub, True)
            for dc in (-1, 0, 1):
                xs = _shift_c(xwh, dc, True)
                bs = _shift_c(bwh, dc, True)
                pair = a8 + (bs >> 1)
                u = jnp.take_along_axis(tab, pair, axis=2)
                odd = (bs & 1) == 1
                bits = jnp.where(odd, u & jnp.int32(-65536), u << 16)
                val = pltpu.bitcast(bits, jnp.float32)
                ft = f_ref[(dc + 1) * 9 + (dh + 1) * 3 + (dw + 1)]
                acc = acc + (ft * xs) * val
    o_ref[0] = acc


def _pack_co_table(co_matrix):
    cb = co_matrix.reshape(-1).astype(jnp.bfloat16)
    u16 = jax.lax.bitcast_convert_type(cb, jnp.uint16).astype(jnp.uint32)
    packed = u16[0::2] | (u16[1::2] << 16)
    return packed.astype(jnp.int32).reshape(1, 128)


def kernel(x, co_matrix, spatial_filter):
    xr = x.reshape(_N * _C * _H, _W)
    mm = pl.pallas_call(
        _minmax_body,
        out_shape=jax.ShapeDtypeStruct((2,), jnp.float32),
        in_specs=[pl.BlockSpec(memory_space=pltpu.VMEM)],
        out_specs=pl.BlockSpec(memory_space=pltpu.SMEM),
    )(xr)

    tab = _pack_co_table(co_matrix)
    f = spatial_filter.reshape(27)

    out = pl.pallas_call(
        _main_body,
        grid=(_N,),
        out_shape=jax.ShapeDtypeStruct((_N, _C, _H, _W), jnp.float32),
        in_specs=[
            pl.BlockSpec(memory_space=pltpu.SMEM),
            pl.BlockSpec(memory_space=pltpu.SMEM),
            pl.BlockSpec((1, 128), lambda n: (0, 0)),
            pl.BlockSpec((1, _C, _H, _W), lambda n: (n, 0, 0, 0)),
        ],
        out_specs=pl.BlockSpec((1, _C, _H, _W), lambda n: (n, 0, 0, 0)),
        compiler_params=pltpu.CompilerParams(
            dimension_semantics=("parallel",),
        ),
    )(mm, f, tab, x)
    return out
